# baseline (speedup 1.0000x reference)
import jax
import jax.numpy as jnp
import numpy as np
from jax.experimental import pallas as pl
from jax.experimental.pallas import tpu as pltpu

NA, NB = 6000, 4000
FT, HID, HID2, OUT = 256, 256, 128, 64
K = 10
BR = 600


def _a_assemble_body(idx_ref, w_ref, out_ref):
    cols = jax.lax.broadcasted_iota(jnp.int32, out_ref.shape, 1)
    acc = jnp.zeros(out_ref.shape, jnp.float32)
    for j in range(K):
        ij = idx_ref[:, j][:, None]
        wj = w_ref[:, j][:, None]
        acc = acc + jnp.where(ij == cols, wj, 0.0)
    out_ref[...] = acc


def _assemble_A(idxa0, w):
    return pl.pallas_call(
        _a_assemble_body,
        grid=(NA // BR,),
        in_specs=[
            pl.BlockSpec((BR, K), lambda i: (i, 0)),
            pl.BlockSpec((BR, K), lambda i: (i, 0)),
        ],
        out_specs=pl.BlockSpec((BR, NA), lambda i: (i, 0)),
        out_shape=jax.ShapeDtypeStruct((NA, NA), jnp.float32),
    )(idxa0, w)


def _mean_agg(feat_src, src, dst, n_dst):
    msg = jnp.take(feat_src, src, axis=0)
    s = jax.ops.segment_sum(msg, dst, num_segments=n_dst)
    cnt = jax.ops.segment_sum(jnp.ones((src.shape[0],), jnp.float32), dst,
                              num_segments=n_dst)
    return s / jnp.maximum(cnt, 1.0)[:, None]


def _spec_mlp(x, W0, b0, W1, b1):
    h = jax.nn.leaky_relu(x @ W0 + b0, negative_slope=0.01)
    return jnp.tanh(h @ W1 + b1)


def kernel(features, features_orth, edge_ab_src, edge_ab_dst, edge_ba_src,
           edge_ba_dst, idx, beta, alpha, W_bnn0_ab, W_bnn0_ba, W_bnn1_ab,
           W_bnn1_ba, W_fc_a, b_fc_a, W_fc_b, b_fc_b, W_sp0, b_sp0, W_sp1,
           b_sp1):
    feat_a = features[:NA]

    embs1_b = jax.nn.relu(_mean_agg(feat_a, edge_ba_src, edge_ba_dst, NB)
                          @ W_bnn0_ba)
    v_a = jax.nn.relu(_mean_agg(embs1_b, edge_ab_src, edge_ab_dst, NA)
                      @ W_bnn1_ab)
    embs_het = v_a @ W_fc_a[:HID2] + feat_a @ W_fc_a[HID2:] + b_fc_a

    Yo = _spec_mlp(features_orth[:NA], W_sp0, b_sp0, W_sp1, b_sp1)
    _, R = jnp.linalg.qr(Yo)
    ow = np.sqrt(NA + 1e-08) * jnp.linalg.inv(R)
    Yt = _spec_mlp(features[:NA], W_sp0, b_sp0, W_sp1, b_sp1)
    Y = Yt @ ow
    Y_2 = Yt

    idxa0 = idx[:, 1:K + 1]
    dfi = jnp.sqrt(jnp.sum((Y[:, None, :] - Y[idxa0]) ** 2, axis=2) + 1e-08)
    ad = -(1.0 + beta[0]) * dfi / (2.0 * alpha[0])

    u = -jnp.sort(-ad, axis=1)
    css = jnp.cumsum(u, axis=1)
    ind = jnp.arange(1, K + 1, dtype=ad.dtype)
    cond = u * ind > (css - 1.0)
    rho = jnp.sum(cond, axis=1).astype(jnp.int32)
    theta = (jnp.take_along_axis(css, (rho - 1)[:, None], axis=1) - 1.0) \
        / rho[:, None].astype(ad.dtype)
    P = jnp.maximum(ad - theta, 0.0)

    eq = idxa0[:, :, None] == idxa0[:, None, :]
    later = jnp.triu(jnp.ones((K, K), bool), k=1)[None]
    dup_later = jnp.any(eq & later, axis=2)
    w = jnp.where(dup_later, 0.0, P)

    A = _assemble_A(idxa0, w)
    embs_hom = jnp.einsum("nk,nkd->nd", w, Y_2[idxa0])
    return (embs_het, embs_hom, A, Y)
